# baseline (speedup 1.0000x reference)
import functools
import jax
import jax.numpy as jnp
from jax import lax
from jax.experimental import pallas as pl
from jax.experimental.pallas import tpu as pltpu

H = 128
NGRID = (8, 8, 8)


def _silu(v):
    return v * jax.nn.sigmoid(v)


def _ln_in(v, g, b):
    m = v.mean(-1, keepdims=True)
    var = v.var(-1, keepdims=True)
    return (v - m) * lax.rsqrt(var + 1e-5) * g + b


def _k1_body(x_ref, rbf_ref, w1_ref, b1_ref, wrbf_ref, g_ref, bln_ref,
             xln_out, t_out):
    x = x_ref[...]
    x_ln = _ln_in(x, g_ref[...], bln_ref[...])
    h = _silu(jnp.dot(x_ln, w1_ref[...], preferred_element_type=jnp.float32)
              + b1_ref[...])
    rbf_h = jnp.dot(rbf_ref[...], wrbf_ref[...],
                    preferred_element_type=jnp.float32)
    xln_out[...] = x_ln
    t_out[...] = h * rbf_h


def _k1(x, rbf, w1, b1, wrbf, g, bln, blk=1000):
    E = x.shape[0]
    grid = (E // blk,)
    return pl.pallas_call(
        _k1_body,
        grid=grid,
        in_specs=[
            pl.BlockSpec((blk, H), lambda i: (i, 0)),
            pl.BlockSpec((blk, 8), lambda i: (i, 0)),
            pl.BlockSpec((H, H), lambda i: (0, 0)),
            pl.BlockSpec((1, H), lambda i: (0, 0)),
            pl.BlockSpec((8, H), lambda i: (0, 0)),
            pl.BlockSpec((1, H), lambda i: (0, 0)),
            pl.BlockSpec((1, H), lambda i: (0, 0)),
        ],
        out_specs=[
            pl.BlockSpec((blk, H), lambda i: (i, 0)),
            pl.BlockSpec((blk, H), lambda i: (i, 0)),
        ],
        out_shape=[
            jax.ShapeDtypeStruct((E, H), jnp.float32),
            jax.ShapeDtypeStruct((E, H), jnp.float32),
        ],
    )(x, rbf, w1, b1, wrbf, g, bln)


def _k2_body(tkj_ref, sbf_ref, wsbf_ref, m_out):
    sbf_h = jnp.dot(sbf_ref[...], wsbf_ref[...],
                    preferred_element_type=jnp.float32)
    m_out[...] = tkj_ref[...] * sbf_h


def _k2(tkj, sbf, wsbf, blk=2000):
    T = sbf.shape[0]
    S = sbf.shape[1]
    return pl.pallas_call(
        _k2_body,
        grid=(T // blk,),
        in_specs=[
            pl.BlockSpec((blk, H), lambda i: (i, 0)),
            pl.BlockSpec((blk, S), lambda i: (i, 0)),
            pl.BlockSpec((S, H), lambda i: (0, 0)),
        ],
        out_specs=pl.BlockSpec((blk, H), lambda i: (i, 0)),
        out_shape=jax.ShapeDtypeStruct((T, H), jnp.float32),
    )(tkj, sbf, wsbf)


def _k3_body(xln_ref, agg_ref, rbf_ref, w2_ref, b2_ref, wlin_ref,
             x2_out, ax_out):
    x2 = _silu(jnp.dot(xln_ref[...] + agg_ref[...], w2_ref[...],
                       preferred_element_type=jnp.float32) + b2_ref[...])
    a_pre = jnp.dot(rbf_ref[...], wlin_ref[...],
                    preferred_element_type=jnp.float32)
    x2_out[...] = x2
    ax_out[...] = a_pre * x2


def _k3(xln, agg, rbf, w2, b2, wlin, blk=1000):
    E = xln.shape[0]
    return pl.pallas_call(
        _k3_body,
        grid=(E // blk,),
        in_specs=[
            pl.BlockSpec((blk, H), lambda i: (i, 0)),
            pl.BlockSpec((blk, H), lambda i: (i, 0)),
            pl.BlockSpec((blk, 8), lambda i: (i, 0)),
            pl.BlockSpec((H, H), lambda i: (0, 0)),
            pl.BlockSpec((1, H), lambda i: (0, 0)),
            pl.BlockSpec((8, H), lambda i: (0, 0)),
        ],
        out_specs=[
            pl.BlockSpec((blk, H), lambda i: (i, 0)),
            pl.BlockSpec((blk, H), lambda i: (i, 0)),
        ],
        out_shape=[
            jax.ShapeDtypeStruct((E, H), jnp.float32),
            jax.ShapeDtypeStruct((E, H), jnp.float32),
        ],
    )(xln, agg, rbf, w2, b2, wlin)


def _k5_body(mx_ref, wr_ref, g_ref, b_ref, out_ref, pad_ref):
    mx = _ln_in(mx_ref[...], g_ref[...], b_ref[...])
    pad_ref[...] = jnp.zeros_like(pad_ref)
    pad_ref[1:9, 1:9, 1:9, :] = mx.reshape(8, 8, 8, H)
    acc = jnp.zeros((512, H), jnp.float32)
    for k in range(27):
        dz, r = divmod(k, 9)
        dy, dx = divmod(r, 3)
        sl = pad_ref[dz:dz + 8, dy:dy + 8, dx:dx + 8, :].reshape(512, H)
        acc = acc + jnp.dot(sl, wr_ref[k], preferred_element_type=jnp.float32)
    out_ref[...] = _silu(acc)


def _k5(m_x, w_r, g, b):
    GM = m_x.shape[0]
    n = GM // 512
    return pl.pallas_call(
        _k5_body,
        grid=(n,),
        in_specs=[
            pl.BlockSpec((512, H), lambda i: (i, 0)),
            pl.BlockSpec((27, H, H), lambda i: (0, 0, 0)),
            pl.BlockSpec((1, H), lambda i: (0, 0)),
            pl.BlockSpec((1, H), lambda i: (0, 0)),
        ],
        out_specs=pl.BlockSpec((512, H), lambda i: (i, 0)),
        out_shape=jax.ShapeDtypeStruct((GM, H), jnp.float32),
        scratch_shapes=[pltpu.VMEM((10, 10, 10, H), jnp.float32)],
    )(m_x, w_r, g, b)


def _k2b_body(ea_a_ref, ea_m_ref, wea_ref, wem_ref, oa_ref, om_ref):
    oa_ref[...] = jnp.dot(ea_a_ref[...], wea_ref[...],
                          preferred_element_type=jnp.float32)
    om_ref[...] = jnp.dot(ea_m_ref[...], wem_ref[...],
                          preferred_element_type=jnp.float32)


def _k2b(ea_a, ea_m, wea, wem, blk=2000):
    EA = ea_a.shape[0]
    R = ea_a.shape[1]
    return pl.pallas_call(
        _k2b_body,
        grid=(EA // blk,),
        in_specs=[
            pl.BlockSpec((blk, R), lambda i: (i, 0)),
            pl.BlockSpec((blk, R), lambda i: (i, 0)),
            pl.BlockSpec((R, H), lambda i: (0, 0)),
            pl.BlockSpec((R, H), lambda i: (0, 0)),
        ],
        out_specs=[
            pl.BlockSpec((blk, H), lambda i: (i, 0)),
            pl.BlockSpec((blk, H), lambda i: (i, 0)),
        ],
        out_shape=[
            jax.ShapeDtypeStruct((EA, H), jnp.float32),
            jax.ShapeDtypeStruct((EA, H), jnp.float32),
        ],
    )(ea_a, ea_m, wea, wem)


def _k6_body(agg_ref, mx2_ref, dmx_ref, w_ref, b_ref, g_ref, bln_ref,
             out_ref):
    msg = _silu(jnp.dot(agg_ref[...], w_ref[...],
                        preferred_element_type=jnp.float32) + b_ref[...])
    msg = _ln_in(msg, g_ref[...], bln_ref[...])
    out_ref[...] = dmx_ref[...] + mx2_ref[...] + msg


def _k6(agg, mx2, dmx, w, b, g, bln, blk=2048):
    GM = mx2.shape[0]
    return pl.pallas_call(
        _k6_body,
        grid=(GM // blk,),
        in_specs=[
            pl.BlockSpec((blk, H), lambda i: (i, 0)),
            pl.BlockSpec((blk, H), lambda i: (i, 0)),
            pl.BlockSpec((blk, H), lambda i: (i, 0)),
            pl.BlockSpec((H, H), lambda i: (0, 0)),
            pl.BlockSpec((1, H), lambda i: (0, 0)),
            pl.BlockSpec((1, H), lambda i: (0, 0)),
            pl.BlockSpec((1, H), lambda i: (0, 0)),
        ],
        out_specs=pl.BlockSpec((blk, H), lambda i: (i, 0)),
        out_shape=jax.ShapeDtypeStruct((GM, H), jnp.float32),
    )(agg, mx2, dmx, w, b, g, bln)


def _k7_body(agg_ref, w_ref, b_ref, out_ref):
    out_ref[...] = _silu(jnp.dot(agg_ref[...], w_ref[...],
                                 preferred_element_type=jnp.float32)
                         + b_ref[...])


def _k7(agg, w, b, blk=1000):
    Nn = agg.shape[0]
    return pl.pallas_call(
        _k7_body,
        grid=(Nn // blk,),
        in_specs=[
            pl.BlockSpec((blk, H), lambda i: (i, 0)),
            pl.BlockSpec((H, H), lambda i: (0, 0)),
            pl.BlockSpec((1, H), lambda i: (0, 0)),
        ],
        out_specs=pl.BlockSpec((blk, H), lambda i: (i, 0)),
        out_shape=jax.ShapeDtypeStruct((Nn, H), jnp.float32),
    )(agg, w, b)


def _k8_body(mj_ref, mi_ref, dx_ref, x2_ref, wt_ref, wb_ref, b_ref, g_ref,
             bln_ref, out_ref):
    v = (jnp.dot(mj_ref[...], wt_ref[...], preferred_element_type=jnp.float32)
         + jnp.dot(mi_ref[...], wb_ref[...],
                   preferred_element_type=jnp.float32) + b_ref[...])
    v = _ln_in(_silu(v), g_ref[...], bln_ref[...])
    out_ref[...] = dx_ref[...] + x2_ref[...] + v


def _k8(mj, mi, dx, x2, wt, wb, b, g, bln, blk=1000):
    E = mj.shape[0]
    return pl.pallas_call(
        _k8_body,
        grid=(E // blk,),
        in_specs=[
            pl.BlockSpec((blk, H), lambda i: (i, 0)),
            pl.BlockSpec((blk, H), lambda i: (i, 0)),
            pl.BlockSpec((blk, H), lambda i: (i, 0)),
            pl.BlockSpec((blk, H), lambda i: (i, 0)),
            pl.BlockSpec((H, H), lambda i: (0, 0)),
            pl.BlockSpec((H, H), lambda i: (0, 0)),
            pl.BlockSpec((1, H), lambda i: (0, 0)),
            pl.BlockSpec((1, H), lambda i: (0, 0)),
            pl.BlockSpec((1, H), lambda i: (0, 0)),
        ],
        out_specs=pl.BlockSpec((blk, H), lambda i: (i, 0)),
        out_shape=jax.ShapeDtypeStruct((E, H), jnp.float32),
    )(mj, mi, dx, x2, wt, wb, b, g, bln)


def kernel(x, rbf, sbf, idx_kj, idx_ji, m_x, a2a_edge_index, a2m_edge_index,
           m2a_edge_index, a2m_edge_weights, m2a_edge_weights, a2m_edge_attr,
           m2a_edge_attr, num_nodes, params):
    p = params
    E = x.shape[0]
    GM = m_x.shape[0]
    N = 10000

    def row(v):
        return v.reshape(1, H)

    x_ln, t = _k1(x, rbf, p['short_W1'], row(p['short_b1']), p['short_W_rbf'],
                  row(p['short_ln_g']), row(p['short_ln_b']))
    tkj = jnp.take(t, idx_kj, axis=0)
    m = _k2(tkj, sbf, p['short_W_sbf'])
    agg = jax.ops.segment_sum(m, idx_ji, num_segments=E)
    x2, ax_edge = _k3(x_ln, agg, rbf, p['short_W2'], row(p['short_b2']),
                      p['lin_rbf_W'])

    a2a_seg = jnp.minimum(a2a_edge_index[1], num_nodes - 1)
    a_x = jax.ops.segment_sum(ax_edge, a2a_seg, num_segments=N)

    w_r = p['conv_W'].transpose(2, 3, 4, 1, 0).reshape(27, H, H)
    mx2 = _k5(m_x, w_r, row(p['long_ln_g']), row(p['long_ln_b']))

    ea_a, ea_m = _k2b(a2m_edge_attr, m2a_edge_attr, p['a2m_We'], p['m2a_We'])

    msg_a = jnp.take(a_x, a2m_edge_index[0], axis=0) \
        * a2m_edge_weights[:, None] + ea_a
    agg_a = jax.ops.segment_sum(msg_a, a2m_edge_index[1], num_segments=GM)
    out_grid = _k6(agg_a, mx2, m_x, p['a2m_W'], row(p['a2m_b']),
                   row(p['a2m_ln_g']), row(p['a2m_ln_b']))

    msg_m = jnp.take(mx2, m2a_edge_index[0], axis=0) \
        * m2a_edge_weights[:, None] + ea_m
    agg_m = jax.ops.segment_sum(msg_m, m2a_edge_index[1], num_segments=N)
    m2a_tab = _k7(agg_m, p['m2a_W'], row(p['m2a_b']))

    mj = jnp.take(m2a_tab, a2a_edge_index[0], axis=0)
    mi = jnp.take(m2a_tab, a2a_edge_index[1], axis=0)
    out_edge = _k8(mj, mi, x, x2, p['comb_W'][:H], p['comb_W'][H:],
                   row(p['comb_b']), row(p['m2a_ln_g']), row(p['m2a_ln_b']))

    return (out_edge, out_grid)
